# baseline (speedup 1.0000x reference)
import math

import jax
import jax.numpy as jnp
from jax import lax
from jax.experimental import pallas as pl
from jax.experimental.pallas import tpu as pltpu
from jax.experimental.pallas import tpu_sc as plsc

_IMG_W = 1333.0
_IMG_H = 800.0
_SCORE_THRESH = 0.05
_NMS_THRESH = 0.5
_DETS = 100
_N = 20000
_C = 81
_CLIP = math.log(1000.0 / 16.0)
_NEG = -1e10

_ROWS = 160
_LANES = 128
_NPAD = _ROWS * _LANES
_TW = 384


_ABLK = 2048
_AGRID = _NPAD // _ABLK
_AR = _ABLK // _LANES


_TBLK = _ABLK * 4 * _C // _LANES


def _score_body(logits_ref, regt_ref, prop_ref,
                score_ref, label_ref, i0_ref, i1_ref, i2_ref, i3_ref,
                tab_ref, p0_ref, p1_ref, p2_ref, p3_ref):
    x = logits_ref[...]
    m = jnp.max(x, axis=1, keepdims=True)
    s = jnp.sum(jnp.exp(x - m), axis=1, keepdims=True)
    score = 1.0 / s
    cols = lax.broadcasted_iota(jnp.int32, x.shape, 1)
    lab = jnp.min(jnp.where(x == m, cols, _C), axis=1, keepdims=True)
    row = (pl.program_id(0) * _ABLK
           + lax.broadcasted_iota(jnp.int32, (_ABLK, 1), 0))
    keep = (lab >= 1) & (score > _SCORE_THRESH) & (row < _N)
    score_ref[...] = jnp.where(keep, score, _NEG).reshape(_AR, _LANES)
    label_ref[...] = lab.reshape(_AR, _LANES)
    f = jnp.where(row < _N,
                  (row >> 11) * (_ABLK * 4 * _C)
                  + ((row & (_ABLK - 1)) >> 7) * (4 * _C * _LANES)
                  + (4 * lab) * _LANES + (row & (_LANES - 1)), 0)
    f10 = f.reshape(_AR, _LANES)
    i0_ref[...] = f10
    i1_ref[...] = f10 + _LANES
    i2_ref[...] = f10 + 2 * _LANES
    i3_ref[...] = f10 + 3 * _LANES
    for j in range(_ABLK // _LANES):
        tab_ref[pl.ds(4 * _C * j, 4 * _C), :] = (
            regt_ref[:, _LANES * j:_LANES * (j + 1)])
    p0_ref[...] = prop_ref[:, 0:1].reshape(_AR, _LANES)
    p1_ref[...] = prop_ref[:, 1:2].reshape(_AR, _LANES)
    p2_ref[...] = prop_ref[:, 2:3].reshape(_AR, _LANES)
    p3_ref[...] = prop_ref[:, 3:4].reshape(_AR, _LANES)


_plane = pl.BlockSpec((_AR, _LANES), lambda i: (i, 0))
_score_call = pl.pallas_call(
    _score_body,
    grid=(_AGRID,),
    in_specs=[
        pl.BlockSpec((_ABLK, _C), lambda i: (i, 0)),
        pl.BlockSpec((4 * _C, _ABLK), lambda i: (0, i)),
        pl.BlockSpec((_ABLK, 4), lambda i: (i, 0)),
    ],
    out_specs=[
        _plane, _plane, _plane, _plane, _plane, _plane,
        pl.BlockSpec((_TBLK, _LANES), lambda i: (i, 0)),
        _plane, _plane, _plane, _plane,
    ],
    out_shape=[
        jax.ShapeDtypeStruct((_ROWS, _LANES), jnp.float32),
        jax.ShapeDtypeStruct((_ROWS, _LANES), jnp.int32),
        jax.ShapeDtypeStruct((_ROWS, _LANES), jnp.int32),
        jax.ShapeDtypeStruct((_ROWS, _LANES), jnp.int32),
        jax.ShapeDtypeStruct((_ROWS, _LANES), jnp.int32),
        jax.ShapeDtypeStruct((_ROWS, _LANES), jnp.int32),
        jax.ShapeDtypeStruct((_AGRID * _TBLK, _LANES), jnp.float32),
        jax.ShapeDtypeStruct((_ROWS, _LANES), jnp.float32),
        jax.ShapeDtypeStruct((_ROWS, _LANES), jnp.float32),
        jax.ShapeDtypeStruct((_ROWS, _LANES), jnp.float32),
        jax.ShapeDtypeStruct((_ROWS, _LANES), jnp.float32),
    ],
)


_info = plsc.get_sparse_core_info()
_NCORE = _info.num_cores
_NSUB = _info.num_subcores
_NW = _NCORE * _NSUB
_CHUNKS = 8
_NWORK = _ROWS // _CHUNKS
_WROWS = 4 * _CHUNKS


def _sc_gather_body(i0, i1, i2, i3, table_hbm, o0, o1, o2, o3,
                    idx_v, ex_v, sem):
    idxs = (i0, i1, i2, i3)
    outs = (o0, o1, o2, o3)
    wid = lax.axis_index("s") * _NCORE + lax.axis_index("c")

    @pl.when(wid < _NWORK)
    def _():
        base = wid * _CHUNKS
        for c in range(4):
            pltpu.sync_copy(idxs[c].at[pl.ds(base, _CHUNKS)],
                            idx_v.at[pl.ds(c * _CHUNKS, _CHUNKS)])
        copies = [
            pltpu.async_copy(table_hbm.at[idx_v.at[t]], ex_v.at[t], sem)
            for t in range(_WROWS)
        ]
        for cp in copies:
            cp.wait()
        for c in range(4):
            pltpu.sync_copy(ex_v.at[pl.ds(c * _CHUNKS, _CHUNKS)],
                            outs[c].at[pl.ds(base, _CHUNKS)])


_gather_call = pl.kernel(
    _sc_gather_body,
    out_type=[jax.ShapeDtypeStruct((_ROWS, _LANES), jnp.float32)
              for _ in range(4)],
    mesh=plsc.VectorSubcoreMesh(core_axis_name="c", subcore_axis_name="s"),
    scratch_types=[
        pltpu.VMEM((_WROWS, _LANES), jnp.int32),
        pltpu.VMEM((_WROWS, _LANES), jnp.float32),
        pltpu.SemaphoreType.DMA,
    ],
)


def _nms_body(score_ref, label_ref, r0_ref, r1_ref, r2_ref, r3_ref,
              p0_ref, p1_ref, p2_ref, p3_ref,
              obox_ref, oscore_ref, olab_ref, sb_ref, sc_ref):
    scores0 = score_ref[...]
    lab_i = label_ref[...]

    px1 = p0_ref[...]
    py1 = p1_ref[...]
    px2 = p2_ref[...]
    py2 = p3_ref[...]
    w = px2 - px1 + 1.0
    h = py2 - py1 + 1.0
    cx = px1 + 0.5 * w
    cy = py1 + 0.5 * h
    dx = r0_ref[...] / 10.0
    dy = r1_ref[...] / 10.0
    dw = jnp.minimum(r2_ref[...] / 5.0, _CLIP)
    dh = jnp.minimum(r3_ref[...] / 5.0, _CLIP)
    pcx = dx * w + cx
    pcy = dy * h + cy
    pw = jnp.exp(dw) * w
    ph = jnp.exp(dh) * h
    bx1 = jnp.clip(pcx - 0.5 * pw, 0.0, _IMG_W - 1.0)
    by1 = jnp.clip(pcy - 0.5 * ph, 0.0, _IMG_H - 1.0)
    bx2 = jnp.clip(pcx + 0.5 * pw - 1.0, 0.0, _IMG_W - 1.0)
    by2 = jnp.clip(pcy + 0.5 * ph - 1.0, 0.0, _IMG_H - 1.0)
    areas = (bx2 - bx1 + 1.0) * (by2 - by1 + 1.0)

    sb_ref[0 * _ROWS:1 * _ROWS, :] = bx1
    sb_ref[1 * _ROWS:2 * _ROWS, :] = by1
    sb_ref[2 * _ROWS:3 * _ROWS, :] = bx2
    sb_ref[3 * _ROWS:4 * _ROWS, :] = by2
    sb_ref[4 * _ROWS:5 * _ROWS, :] = areas
    sb_ref[5 * _ROWS:6 * _ROWS, :] = lab_i.astype(jnp.float32)

    sc_ref[...] = scores0
    rowi = lax.broadcasted_iota(jnp.int32, (_ROWS, _LANES), 0)
    coli = lax.broadcasted_iota(jnp.int32, (_ROWS, _LANES), 1)
    riota = lax.broadcasted_iota(jnp.int32, (_ROWS, 1), 0)
    col = lax.broadcasted_iota(jnp.int32, (1, _LANES), 1)
    zrow = jnp.zeros((1, _LANES), jnp.float32)

    def step(i, carry):
        os_, ox1, oy1, ox2, oy2, ol = carry
        scores = sc_ref[...]
        rm = jnp.max(scores, axis=1, keepdims=True)
        gm = jnp.max(rm)
        br = jnp.min(jnp.where(rm == gm, riota, _ROWS))
        srow = sc_ref[pl.ds(br, 1), :]
        bc = jnp.min(jnp.where(srow == gm, col, _LANES))
        isb = (rowi == br) & (coli == bc)
        cm = (col == bc).astype(jnp.float32)
        sx1 = jnp.sum(sb_ref[pl.ds(0 * _ROWS + br, 1), :] * cm)
        sy1 = jnp.sum(sb_ref[pl.ds(1 * _ROWS + br, 1), :] * cm)
        sx2 = jnp.sum(sb_ref[pl.ds(2 * _ROWS + br, 1), :] * cm)
        sy2 = jnp.sum(sb_ref[pl.ds(3 * _ROWS + br, 1), :] * cm)
        sarea = jnp.sum(sb_ref[pl.ds(4 * _ROWS + br, 1), :] * cm)
        slab = jnp.sum(sb_ref[pl.ds(5 * _ROWS + br, 1), :] * cm)
        xx1 = jnp.maximum(sx1, bx1)
        yy1 = jnp.maximum(sy1, by1)
        xx2 = jnp.minimum(sx2, bx2)
        yy2 = jnp.minimum(sy2, by2)
        inter = (jnp.maximum(xx2 - xx1 + 1.0, 0.0)
                 * jnp.maximum(yy2 - yy1 + 1.0, 0.0))
        iou = inter / (sarea + areas - inter)
        sc_ref[...] = jnp.where((iou > _NMS_THRESH) | isb, _NEG, scores)
        valid = gm > 0.0
        vf = jnp.where(valid, 1.0, 0.0)
        hit = col == i
        os_ = jnp.where(hit, gm * vf, os_)
        ox1 = jnp.where(hit, sx1 * vf, ox1)
        oy1 = jnp.where(hit, sy1 * vf, oy1)
        ox2 = jnp.where(hit, sx2 * vf, ox2)
        oy2 = jnp.where(hit, sy2 * vf, oy2)
        ol = jnp.where(hit, slab * vf, ol)
        return os_, ox1, oy1, ox2, oy2, ol

    init = (zrow, zrow, zrow, zrow, zrow, zrow)
    os_, ox1, oy1, ox2, oy2, ol = lax.fori_loop(0, _DETS, step, init)
    obox_ref[0:1, :] = ox1
    obox_ref[1:2, :] = oy1
    obox_ref[2:3, :] = ox2
    obox_ref[3:4, :] = oy2
    oscore_ref[...] = os_
    olab_ref[...] = (ol + 0.5).astype(jnp.int32)


_nms_call = pl.pallas_call(
    _nms_body,
    out_shape=[
        jax.ShapeDtypeStruct((4, _LANES), jnp.float32),
        jax.ShapeDtypeStruct((1, _LANES), jnp.float32),
        jax.ShapeDtypeStruct((1, _LANES), jnp.int32),
    ],
    scratch_shapes=[pltpu.VMEM((6 * _ROWS, _LANES), jnp.float32),
                    pltpu.VMEM((_ROWS, _LANES), jnp.float32)],
)


@jax.jit
def kernel(class_logits, box_regression, proposal_boxes):
    (score, label, i0, i1, i2, i3, table,
     p0, p1, p2, p3) = _score_call(class_logits, box_regression.T,
                                   proposal_boxes)
    r0, r1, r2, r3 = _gather_call(i0, i1, i2, i3, table.reshape(-1))
    obox, oscore, olab = _nms_call(score, label, r0, r1, r2, r3,
                                   p0, p1, p2, p3)
    return obox[:, :_DETS].T, oscore[0, :_DETS], olab[0, :_DETS]


# speedup vs baseline: 1.0462x; 1.0462x over previous
import math

import jax
import jax.numpy as jnp
from jax import lax
from jax.experimental import pallas as pl
from jax.experimental.pallas import tpu as pltpu
from jax.experimental.pallas import tpu_sc as plsc

_IMG_W = 1333.0
_IMG_H = 800.0
_SCORE_THRESH = 0.05
_NMS_THRESH = 0.5
_DETS = 100
_N = 20000
_C = 81
_CLIP = math.log(1000.0 / 16.0)
_NEG = -1e10

_ROWS = 160
_LANES = 128
_NPAD = _ROWS * _LANES
_TW = 384


_ABLK = 2048
_AGRID = _NPAD // _ABLK
_AR = _ABLK // _LANES


_TBLK = _ABLK * 4 * _C // _LANES


def _score_body(logits_ref, regt_ref, prop_ref,
                score_ref, label_ref, i0_ref, i1_ref, i2_ref, i3_ref,
                tab_ref, p0_ref, p1_ref, p2_ref, p3_ref):
    x = logits_ref[...]
    m = jnp.max(x, axis=1, keepdims=True)
    s = jnp.sum(jnp.exp(x - m), axis=1, keepdims=True)
    score = 1.0 / s
    cols = lax.broadcasted_iota(jnp.int32, x.shape, 1)
    lab = jnp.min(jnp.where(x == m, cols, _C), axis=1, keepdims=True)
    row = (pl.program_id(0) * _ABLK
           + lax.broadcasted_iota(jnp.int32, (_ABLK, 1), 0))
    keep = (lab >= 1) & (score > _SCORE_THRESH) & (row < _N)
    score_ref[...] = jnp.where(keep, score, _NEG).reshape(_AR, _LANES)
    label_ref[...] = lab.reshape(_AR, _LANES)
    f = jnp.where(row < _N,
                  (row >> 11) * (_ABLK * 4 * _C)
                  + ((row & (_ABLK - 1)) >> 7) * (4 * _C * _LANES)
                  + (4 * lab) * _LANES + (row & (_LANES - 1)), 0)
    f10 = f.reshape(_AR, _LANES)
    i0_ref[...] = f10
    i1_ref[...] = f10 + _LANES
    i2_ref[...] = f10 + 2 * _LANES
    i3_ref[...] = f10 + 3 * _LANES
    for j in range(_ABLK // _LANES):
        tab_ref[pl.ds(4 * _C * j, 4 * _C), :] = (
            regt_ref[:, _LANES * j:_LANES * (j + 1)])
    p0_ref[...] = prop_ref[:, 0:1].reshape(_AR, _LANES)
    p1_ref[...] = prop_ref[:, 1:2].reshape(_AR, _LANES)
    p2_ref[...] = prop_ref[:, 2:3].reshape(_AR, _LANES)
    p3_ref[...] = prop_ref[:, 3:4].reshape(_AR, _LANES)


_plane = pl.BlockSpec((_AR, _LANES), lambda i: (i, 0))
_score_call = pl.pallas_call(
    _score_body,
    grid=(_AGRID,),
    in_specs=[
        pl.BlockSpec((_ABLK, _C), lambda i: (i, 0)),
        pl.BlockSpec((4 * _C, _ABLK), lambda i: (0, i)),
        pl.BlockSpec((_ABLK, 4), lambda i: (i, 0)),
    ],
    out_specs=[
        _plane, _plane, _plane, _plane, _plane, _plane,
        pl.BlockSpec((_TBLK, _LANES), lambda i: (i, 0)),
        _plane, _plane, _plane, _plane,
    ],
    out_shape=[
        jax.ShapeDtypeStruct((_ROWS, _LANES), jnp.float32),
        jax.ShapeDtypeStruct((_ROWS, _LANES), jnp.int32),
        jax.ShapeDtypeStruct((_ROWS, _LANES), jnp.int32),
        jax.ShapeDtypeStruct((_ROWS, _LANES), jnp.int32),
        jax.ShapeDtypeStruct((_ROWS, _LANES), jnp.int32),
        jax.ShapeDtypeStruct((_ROWS, _LANES), jnp.int32),
        jax.ShapeDtypeStruct((_AGRID * _TBLK, _LANES), jnp.float32),
        jax.ShapeDtypeStruct((_ROWS, _LANES), jnp.float32),
        jax.ShapeDtypeStruct((_ROWS, _LANES), jnp.float32),
        jax.ShapeDtypeStruct((_ROWS, _LANES), jnp.float32),
        jax.ShapeDtypeStruct((_ROWS, _LANES), jnp.float32),
    ],
)


_info = plsc.get_sparse_core_info()
_NCORE = _info.num_cores
_NSUB = _info.num_subcores
_NW = _NCORE * _NSUB
_CHUNKS = 8
_NWORK = _ROWS // _CHUNKS
_WROWS = 4 * _CHUNKS


def _sc_gather_body(i0, i1, i2, i3, table_hbm, o0, o1, o2, o3,
                    idx_v, ex_v, sem):
    idxs = (i0, i1, i2, i3)
    outs = (o0, o1, o2, o3)
    wid = lax.axis_index("s") * _NCORE + lax.axis_index("c")

    @pl.when(wid < _NWORK)
    def _():
        base = wid * _CHUNKS
        for c in range(4):
            pltpu.sync_copy(idxs[c].at[pl.ds(base, _CHUNKS)],
                            idx_v.at[pl.ds(c * _CHUNKS, _CHUNKS)])
        copies = [
            pltpu.async_copy(table_hbm.at[idx_v.at[t]], ex_v.at[t], sem)
            for t in range(_WROWS)
        ]
        for cp in copies:
            cp.wait()
        for c in range(4):
            pltpu.sync_copy(ex_v.at[pl.ds(c * _CHUNKS, _CHUNKS)],
                            outs[c].at[pl.ds(base, _CHUNKS)])


_gather_call = pl.kernel(
    _sc_gather_body,
    out_type=[jax.ShapeDtypeStruct((_ROWS, _LANES), jnp.float32)
              for _ in range(4)],
    mesh=plsc.VectorSubcoreMesh(core_axis_name="c", subcore_axis_name="s"),
    scratch_types=[
        pltpu.VMEM((_WROWS, _LANES), jnp.int32),
        pltpu.VMEM((_WROWS, _LANES), jnp.float32),
        pltpu.SemaphoreType.DMA,
    ],
)


def _nms_body(score_ref, label_ref, r0_ref, r1_ref, r2_ref, r3_ref,
              p0_ref, p1_ref, p2_ref, p3_ref,
              obox_ref, oscore_ref, olab_ref, sb_ref):
    scores0 = score_ref[...]
    lab_i = label_ref[...]

    px1 = p0_ref[...]
    py1 = p1_ref[...]
    px2 = p2_ref[...]
    py2 = p3_ref[...]
    w = px2 - px1 + 1.0
    h = py2 - py1 + 1.0
    cx = px1 + 0.5 * w
    cy = py1 + 0.5 * h
    dx = r0_ref[...] / 10.0
    dy = r1_ref[...] / 10.0
    dw = jnp.minimum(r2_ref[...] / 5.0, _CLIP)
    dh = jnp.minimum(r3_ref[...] / 5.0, _CLIP)
    pcx = dx * w + cx
    pcy = dy * h + cy
    pw = jnp.exp(dw) * w
    ph = jnp.exp(dh) * h
    bx1 = jnp.clip(pcx - 0.5 * pw, 0.0, _IMG_W - 1.0)
    by1 = jnp.clip(pcy - 0.5 * ph, 0.0, _IMG_H - 1.0)
    bx2 = jnp.clip(pcx + 0.5 * pw - 1.0, 0.0, _IMG_W - 1.0)
    by2 = jnp.clip(pcy + 0.5 * ph - 1.0, 0.0, _IMG_H - 1.0)
    areas = (bx2 - bx1 + 1.0) * (by2 - by1 + 1.0)

    sb_ref[0 * _ROWS:1 * _ROWS, :] = bx1
    sb_ref[1 * _ROWS:2 * _ROWS, :] = by1
    sb_ref[2 * _ROWS:3 * _ROWS, :] = bx2
    sb_ref[3 * _ROWS:4 * _ROWS, :] = by2
    sb_ref[4 * _ROWS:5 * _ROWS, :] = areas
    sb_ref[5 * _ROWS:6 * _ROWS, :] = lab_i.astype(jnp.float32)

    flat = (lax.broadcasted_iota(jnp.int32, (_ROWS, _LANES), 0) * _LANES
            + lax.broadcasted_iota(jnp.int32, (_ROWS, _LANES), 1))
    col = lax.broadcasted_iota(jnp.int32, (1, _LANES), 1)
    zrow = jnp.zeros((1, _LANES), jnp.float32)

    def step(i, carry):
        scores, os_, ox1, oy1, ox2, oy2, ol = carry
        gm = jnp.max(scores)
        bf = jnp.min(jnp.where(scores == gm, flat, jnp.int32(2147483647)))
        isb = flat == bf
        br = bf >> 7
        cm = (col == (bf & 127)).astype(jnp.float32)
        sx1 = jnp.sum(sb_ref[pl.ds(0 * _ROWS + br, 1), :] * cm)
        sy1 = jnp.sum(sb_ref[pl.ds(1 * _ROWS + br, 1), :] * cm)
        sx2 = jnp.sum(sb_ref[pl.ds(2 * _ROWS + br, 1), :] * cm)
        sy2 = jnp.sum(sb_ref[pl.ds(3 * _ROWS + br, 1), :] * cm)
        sarea = jnp.sum(sb_ref[pl.ds(4 * _ROWS + br, 1), :] * cm)
        slab = jnp.sum(sb_ref[pl.ds(5 * _ROWS + br, 1), :] * cm)
        xx1 = jnp.maximum(sx1, bx1)
        yy1 = jnp.maximum(sy1, by1)
        xx2 = jnp.minimum(sx2, bx2)
        yy2 = jnp.minimum(sy2, by2)
        inter = (jnp.maximum(xx2 - xx1 + 1.0, 0.0)
                 * jnp.maximum(yy2 - yy1 + 1.0, 0.0))
        iou = inter / (sarea + areas - inter)
        scores = jnp.where((iou > _NMS_THRESH) | isb, _NEG, scores)
        valid = gm > 0.0
        vf = jnp.where(valid, 1.0, 0.0)
        hit = col == i
        os_ = jnp.where(hit, gm * vf, os_)
        ox1 = jnp.where(hit, sx1 * vf, ox1)
        oy1 = jnp.where(hit, sy1 * vf, oy1)
        ox2 = jnp.where(hit, sx2 * vf, ox2)
        oy2 = jnp.where(hit, sy2 * vf, oy2)
        ol = jnp.where(hit, slab * vf, ol)
        return scores, os_, ox1, oy1, ox2, oy2, ol

    init = (scores0, zrow, zrow, zrow, zrow, zrow, zrow)
    _, os_, ox1, oy1, ox2, oy2, ol = lax.fori_loop(0, _DETS, step, init)
    obox_ref[0:1, :] = ox1
    obox_ref[1:2, :] = oy1
    obox_ref[2:3, :] = ox2
    obox_ref[3:4, :] = oy2
    oscore_ref[...] = os_
    olab_ref[...] = (ol + 0.5).astype(jnp.int32)


_nms_call = pl.pallas_call(
    _nms_body,
    out_shape=[
        jax.ShapeDtypeStruct((4, _LANES), jnp.float32),
        jax.ShapeDtypeStruct((1, _LANES), jnp.float32),
        jax.ShapeDtypeStruct((1, _LANES), jnp.int32),
    ],
    scratch_shapes=[pltpu.VMEM((6 * _ROWS, _LANES), jnp.float32)],
)


@jax.jit
def kernel(class_logits, box_regression, proposal_boxes):
    (score, label, i0, i1, i2, i3, table,
     p0, p1, p2, p3) = _score_call(class_logits, box_regression.T,
                                   proposal_boxes)
    r0, r1, r2, r3 = _gather_call(i0, i1, i2, i3, table.reshape(-1))
    obox, oscore, olab = _nms_call(score, label, r0, r1, r2, r3,
                                   p0, p1, p2, p3)
    return obox[:, :_DETS].T, oscore[0, :_DETS], olab[0, :_DETS]


# speedup vs baseline: 1.3355x; 1.2765x over previous
import math

import jax
import jax.numpy as jnp
from jax import lax
from jax.experimental import pallas as pl
from jax.experimental.pallas import tpu as pltpu
from jax.experimental.pallas import tpu_sc as plsc

_IMG_W = 1333.0
_IMG_H = 800.0
_SCORE_THRESH = 0.05
_NMS_THRESH = 0.5
_DETS = 100
_N = 20000
_C = 81
_CLIP = math.log(1000.0 / 16.0)
_NEG = -1e10

_ROWS = 160
_LANES = 128
_NPAD = _ROWS * _LANES
_TW = 384


_ABLK = 2048
_AGRID = _NPAD // _ABLK
_AR = _ABLK // _LANES


_TBLK = _ABLK * 4 * _C // _LANES


def _score_body(logits_ref, regt_ref, prop_ref,
                score_ref, label_ref, i0_ref, i1_ref, i2_ref, i3_ref,
                tab_ref, p0_ref, p1_ref, p2_ref, p3_ref):
    x = logits_ref[...]
    m = jnp.max(x, axis=0, keepdims=True)
    s = jnp.sum(jnp.exp(x - m), axis=0, keepdims=True)
    score = 1.0 / s
    cls = lax.broadcasted_iota(jnp.int32, x.shape, 0)
    lab = jnp.min(jnp.where(x == m, cls, _C), axis=0, keepdims=True)
    row = (pl.program_id(0) * _ABLK
           + lax.broadcasted_iota(jnp.int32, (1, _ABLK), 1))
    keep = (lab >= 1) & (score > _SCORE_THRESH) & (row < _N)
    score_ref[...] = jnp.where(keep, score, _NEG).reshape(_AR, _LANES)
    label_ref[...] = lab.reshape(_AR, _LANES)
    f = jnp.where(row < _N,
                  (row >> 11) * (_ABLK * 4 * _C)
                  + ((row & (_ABLK - 1)) >> 7) * (4 * _C * _LANES)
                  + (4 * lab) * _LANES + (row & (_LANES - 1)), 0)
    f10 = f.reshape(_AR, _LANES)
    i0_ref[...] = f10
    i1_ref[...] = f10 + _LANES
    i2_ref[...] = f10 + 2 * _LANES
    i3_ref[...] = f10 + 3 * _LANES
    for j in range(_ABLK // _LANES):
        tab_ref[pl.ds(4 * _C * j, 4 * _C), :] = (
            regt_ref[:, _LANES * j:_LANES * (j + 1)])
    p0_ref[...] = prop_ref[0:1, :].reshape(_AR, _LANES)
    p1_ref[...] = prop_ref[1:2, :].reshape(_AR, _LANES)
    p2_ref[...] = prop_ref[2:3, :].reshape(_AR, _LANES)
    p3_ref[...] = prop_ref[3:4, :].reshape(_AR, _LANES)


_plane = pl.BlockSpec((_AR, _LANES), lambda i: (i, 0))
_score_call = pl.pallas_call(
    _score_body,
    grid=(_AGRID,),
    in_specs=[
        pl.BlockSpec((_C, _ABLK), lambda i: (0, i)),
        pl.BlockSpec((4 * _C, _ABLK), lambda i: (0, i)),
        pl.BlockSpec((4, _ABLK), lambda i: (0, i)),
    ],
    out_specs=[
        _plane, _plane, _plane, _plane, _plane, _plane,
        pl.BlockSpec((_TBLK, _LANES), lambda i: (i, 0)),
        _plane, _plane, _plane, _plane,
    ],
    out_shape=[
        jax.ShapeDtypeStruct((_ROWS, _LANES), jnp.float32),
        jax.ShapeDtypeStruct((_ROWS, _LANES), jnp.int32),
        jax.ShapeDtypeStruct((_ROWS, _LANES), jnp.int32),
        jax.ShapeDtypeStruct((_ROWS, _LANES), jnp.int32),
        jax.ShapeDtypeStruct((_ROWS, _LANES), jnp.int32),
        jax.ShapeDtypeStruct((_ROWS, _LANES), jnp.int32),
        jax.ShapeDtypeStruct((_AGRID * _TBLK, _LANES), jnp.float32),
        jax.ShapeDtypeStruct((_ROWS, _LANES), jnp.float32),
        jax.ShapeDtypeStruct((_ROWS, _LANES), jnp.float32),
        jax.ShapeDtypeStruct((_ROWS, _LANES), jnp.float32),
        jax.ShapeDtypeStruct((_ROWS, _LANES), jnp.float32),
    ],
)


_info = plsc.get_sparse_core_info()
_NCORE = _info.num_cores
_NSUB = _info.num_subcores
_NW = _NCORE * _NSUB
_CHUNKS = 8
_NWORK = _ROWS // _CHUNKS
_WROWS = 4 * _CHUNKS


def _sc_gather_body(i0, i1, i2, i3, table_hbm, o0, o1, o2, o3,
                    idx_v, ex_v, sem):
    idxs = (i0, i1, i2, i3)
    outs = (o0, o1, o2, o3)
    wid = lax.axis_index("s") * _NCORE + lax.axis_index("c")

    @pl.when(wid < _NWORK)
    def _():
        base = wid * _CHUNKS
        for c in range(4):
            pltpu.sync_copy(idxs[c].at[pl.ds(base, _CHUNKS)],
                            idx_v.at[pl.ds(c * _CHUNKS, _CHUNKS)])
        copies = [
            pltpu.async_copy(table_hbm.at[idx_v.at[t]], ex_v.at[t], sem)
            for t in range(_WROWS)
        ]
        for cp in copies:
            cp.wait()
        for c in range(4):
            pltpu.sync_copy(ex_v.at[pl.ds(c * _CHUNKS, _CHUNKS)],
                            outs[c].at[pl.ds(base, _CHUNKS)])


_gather_call = pl.kernel(
    _sc_gather_body,
    out_type=[jax.ShapeDtypeStruct((_ROWS, _LANES), jnp.float32)
              for _ in range(4)],
    mesh=plsc.VectorSubcoreMesh(core_axis_name="c", subcore_axis_name="s"),
    scratch_types=[
        pltpu.VMEM((_WROWS, _LANES), jnp.int32),
        pltpu.VMEM((_WROWS, _LANES), jnp.float32),
        pltpu.SemaphoreType.DMA,
    ],
)


def _nms_body(score_ref, label_ref, r0_ref, r1_ref, r2_ref, r3_ref,
              p0_ref, p1_ref, p2_ref, p3_ref,
              obox_ref, oscore_ref, olab_ref, sb_ref):
    scores0 = score_ref[...]
    lab_i = label_ref[...]

    px1 = p0_ref[...]
    py1 = p1_ref[...]
    px2 = p2_ref[...]
    py2 = p3_ref[...]
    w = px2 - px1 + 1.0
    h = py2 - py1 + 1.0
    cx = px1 + 0.5 * w
    cy = py1 + 0.5 * h
    dx = r0_ref[...] / 10.0
    dy = r1_ref[...] / 10.0
    dw = jnp.minimum(r2_ref[...] / 5.0, _CLIP)
    dh = jnp.minimum(r3_ref[...] / 5.0, _CLIP)
    pcx = dx * w + cx
    pcy = dy * h + cy
    pw = jnp.exp(dw) * w
    ph = jnp.exp(dh) * h
    bx1 = jnp.clip(pcx - 0.5 * pw, 0.0, _IMG_W - 1.0)
    by1 = jnp.clip(pcy - 0.5 * ph, 0.0, _IMG_H - 1.0)
    bx2 = jnp.clip(pcx + 0.5 * pw - 1.0, 0.0, _IMG_W - 1.0)
    by2 = jnp.clip(pcy + 0.5 * ph - 1.0, 0.0, _IMG_H - 1.0)
    areas = (bx2 - bx1 + 1.0) * (by2 - by1 + 1.0)

    sb_ref[0 * _ROWS:1 * _ROWS, :] = bx1
    sb_ref[1 * _ROWS:2 * _ROWS, :] = by1
    sb_ref[2 * _ROWS:3 * _ROWS, :] = bx2
    sb_ref[3 * _ROWS:4 * _ROWS, :] = by2
    sb_ref[4 * _ROWS:5 * _ROWS, :] = areas
    sb_ref[5 * _ROWS:6 * _ROWS, :] = lab_i.astype(jnp.float32)

    flat = (lax.broadcasted_iota(jnp.int32, (_ROWS, _LANES), 0) * _LANES
            + lax.broadcasted_iota(jnp.int32, (_ROWS, _LANES), 1))
    col = lax.broadcasted_iota(jnp.int32, (1, _LANES), 1)
    zrow = jnp.zeros((1, _LANES), jnp.float32)

    def step(i, carry):
        scores, os_, ox1, oy1, ox2, oy2, ol = carry
        gm = jnp.max(scores)
        bf = jnp.min(jnp.where(scores == gm, flat, jnp.int32(2147483647)))
        isb = flat == bf
        br = bf >> 7
        cm = (col == (bf & 127)).astype(jnp.float32)
        sx1 = jnp.sum(sb_ref[pl.ds(0 * _ROWS + br, 1), :] * cm)
        sy1 = jnp.sum(sb_ref[pl.ds(1 * _ROWS + br, 1), :] * cm)
        sx2 = jnp.sum(sb_ref[pl.ds(2 * _ROWS + br, 1), :] * cm)
        sy2 = jnp.sum(sb_ref[pl.ds(3 * _ROWS + br, 1), :] * cm)
        sarea = jnp.sum(sb_ref[pl.ds(4 * _ROWS + br, 1), :] * cm)
        slab = jnp.sum(sb_ref[pl.ds(5 * _ROWS + br, 1), :] * cm)
        xx1 = jnp.maximum(sx1, bx1)
        yy1 = jnp.maximum(sy1, by1)
        xx2 = jnp.minimum(sx2, bx2)
        yy2 = jnp.minimum(sy2, by2)
        inter = (jnp.maximum(xx2 - xx1 + 1.0, 0.0)
                 * jnp.maximum(yy2 - yy1 + 1.0, 0.0))
        iou = inter / (sarea + areas - inter)
        scores = jnp.where((iou > _NMS_THRESH) | isb, _NEG, scores)
        valid = gm > 0.0
        vf = jnp.where(valid, 1.0, 0.0)
        hit = col == i
        os_ = jnp.where(hit, gm * vf, os_)
        ox1 = jnp.where(hit, sx1 * vf, ox1)
        oy1 = jnp.where(hit, sy1 * vf, oy1)
        ox2 = jnp.where(hit, sx2 * vf, ox2)
        oy2 = jnp.where(hit, sy2 * vf, oy2)
        ol = jnp.where(hit, slab * vf, ol)
        return scores, os_, ox1, oy1, ox2, oy2, ol

    init = (scores0, zrow, zrow, zrow, zrow, zrow, zrow)
    _, os_, ox1, oy1, ox2, oy2, ol = lax.fori_loop(0, _DETS, step, init)
    obox_ref[0:1, :] = ox1
    obox_ref[1:2, :] = oy1
    obox_ref[2:3, :] = ox2
    obox_ref[3:4, :] = oy2
    oscore_ref[...] = os_
    olab_ref[...] = (ol + 0.5).astype(jnp.int32)


_nms_call = pl.pallas_call(
    _nms_body,
    out_shape=[
        jax.ShapeDtypeStruct((4, _LANES), jnp.float32),
        jax.ShapeDtypeStruct((1, _LANES), jnp.float32),
        jax.ShapeDtypeStruct((1, _LANES), jnp.int32),
    ],
    scratch_shapes=[pltpu.VMEM((6 * _ROWS, _LANES), jnp.float32)],
)


@jax.jit
def kernel(class_logits, box_regression, proposal_boxes):
    (score, label, i0, i1, i2, i3, table,
     p0, p1, p2, p3) = _score_call(class_logits.T, box_regression.T,
                                   proposal_boxes.T)
    r0, r1, r2, r3 = _gather_call(i0, i1, i2, i3, table.reshape(-1))
    obox, oscore, olab = _nms_call(score, label, r0, r1, r2, r3,
                                   p0, p1, p2, p3)
    return obox[:, :_DETS].T, oscore[0, :_DETS], olab[0, :_DETS]


# speedup vs baseline: 1.3356x; 1.0001x over previous
import math

import jax
import jax.numpy as jnp
from jax import lax
from jax.experimental import pallas as pl
from jax.experimental.pallas import tpu as pltpu
from jax.experimental.pallas import tpu_sc as plsc

_IMG_W = 1333.0
_IMG_H = 800.0
_SCORE_THRESH = 0.05
_NMS_THRESH = 0.5
_DETS = 100
_N = 20000
_C = 81
_CLIP = math.log(1000.0 / 16.0)
_NEG = -1e10

_ROWS = 160
_LANES = 128
_NPAD = _ROWS * _LANES


_ABLK = 2048
_AGRID = _NPAD // _ABLK
_AR = _ABLK // _LANES
_TBLK = _ABLK * 4 * _C // _LANES


def _score_body(logits_ref, regt_ref, prop_ref,
                score_ref, label_ref, i0_ref, i1_ref, i2_ref, i3_ref,
                tab_ref, p0_ref, p1_ref, p2_ref, p3_ref):
    x = logits_ref[...]
    m = jnp.max(x, axis=0, keepdims=True)
    s = jnp.sum(jnp.exp(x - m), axis=0, keepdims=True)
    score = 1.0 / s
    cls = lax.broadcasted_iota(jnp.int32, x.shape, 0)
    lab = jnp.min(jnp.where(x == m, cls, _C), axis=0, keepdims=True)
    row = (pl.program_id(0) * _ABLK
           + lax.broadcasted_iota(jnp.int32, (1, _ABLK), 1))
    keep = (lab >= 1) & (score > _SCORE_THRESH) & (row < _N)
    score_ref[...] = jnp.where(keep, score, _NEG).reshape(_AR, _LANES)
    label_ref[...] = lab.reshape(_AR, _LANES)
    f = jnp.where(row < _N,
                  (row >> 11) * (_ABLK * 4 * _C)
                  + ((row & (_ABLK - 1)) >> 7) * (4 * _C * _LANES)
                  + (4 * lab) * _LANES + (row & (_LANES - 1)), 0)
    f10 = f.reshape(_AR, _LANES)
    i0_ref[...] = f10
    i1_ref[...] = f10 + _LANES
    i2_ref[...] = f10 + 2 * _LANES
    i3_ref[...] = f10 + 3 * _LANES
    for j in range(_ABLK // _LANES):
        tab_ref[pl.ds(4 * _C * j, 4 * _C), :] = (
            regt_ref[:, _LANES * j:_LANES * (j + 1)])
    p0_ref[...] = prop_ref[0:1, :].reshape(_AR, _LANES)
    p1_ref[...] = prop_ref[1:2, :].reshape(_AR, _LANES)
    p2_ref[...] = prop_ref[2:3, :].reshape(_AR, _LANES)
    p3_ref[...] = prop_ref[3:4, :].reshape(_AR, _LANES)


_plane = pl.BlockSpec((_AR, _LANES), lambda i: (i, 0))
_score_call = pl.pallas_call(
    _score_body,
    grid=(_AGRID,),
    in_specs=[
        pl.BlockSpec((_C, _ABLK), lambda i: (0, i)),
        pl.BlockSpec((4 * _C, _ABLK), lambda i: (0, i)),
        pl.BlockSpec((4, _ABLK), lambda i: (0, i)),
    ],
    out_specs=[
        _plane, _plane, _plane, _plane, _plane, _plane,
        pl.BlockSpec((_TBLK, _LANES), lambda i: (i, 0)),
        _plane, _plane, _plane, _plane,
    ],
    out_shape=[
        jax.ShapeDtypeStruct((_ROWS, _LANES), jnp.float32),
        jax.ShapeDtypeStruct((_ROWS, _LANES), jnp.int32),
        jax.ShapeDtypeStruct((_ROWS, _LANES), jnp.int32),
        jax.ShapeDtypeStruct((_ROWS, _LANES), jnp.int32),
        jax.ShapeDtypeStruct((_ROWS, _LANES), jnp.int32),
        jax.ShapeDtypeStruct((_ROWS, _LANES), jnp.int32),
        jax.ShapeDtypeStruct((_AGRID * _TBLK, _LANES), jnp.float32),
        jax.ShapeDtypeStruct((_ROWS, _LANES), jnp.float32),
        jax.ShapeDtypeStruct((_ROWS, _LANES), jnp.float32),
        jax.ShapeDtypeStruct((_ROWS, _LANES), jnp.float32),
        jax.ShapeDtypeStruct((_ROWS, _LANES), jnp.float32),
    ],
)


_info = plsc.get_sparse_core_info()
_NCORE = _info.num_cores
_NSUB = _info.num_subcores
_NW = _NCORE * _NSUB
_CHUNKS = 8
_NWORK = _ROWS // _CHUNKS
_WROWS = 4 * _CHUNKS


def _sc_gather_body(i0, i1, i2, i3, table_hbm, o0, o1, o2, o3,
                    idx_v, ex_v, sem):
    idxs = (i0, i1, i2, i3)
    outs = (o0, o1, o2, o3)
    wid = lax.axis_index("s") * _NCORE + lax.axis_index("c")

    @pl.when(wid < _NWORK)
    def _():
        base = wid * _CHUNKS
        for c in range(4):
            pltpu.sync_copy(idxs[c].at[pl.ds(base, _CHUNKS)],
                            idx_v.at[pl.ds(c * _CHUNKS, _CHUNKS)])
        copies = [
            pltpu.async_copy(table_hbm.at[idx_v.at[t]], ex_v.at[t], sem)
            for t in range(_WROWS)
        ]
        for cp in copies:
            cp.wait()
        for c in range(4):
            pltpu.sync_copy(ex_v.at[pl.ds(c * _CHUNKS, _CHUNKS)],
                            outs[c].at[pl.ds(base, _CHUNKS)])


_gather_call = pl.kernel(
    _sc_gather_body,
    out_type=[jax.ShapeDtypeStruct((_ROWS, _LANES), jnp.float32)
              for _ in range(4)],
    mesh=plsc.VectorSubcoreMesh(core_axis_name="c", subcore_axis_name="s"),
    scratch_types=[
        pltpu.VMEM((_WROWS, _LANES), jnp.int32),
        pltpu.VMEM((_WROWS, _LANES), jnp.float32),
        pltpu.SemaphoreType.DMA,
    ],
)


def _nms_body(score_ref, label_ref, r0_ref, r1_ref, r2_ref, r3_ref,
              p0_ref, p1_ref, p2_ref, p3_ref,
              obox_ref, oscore_ref, olab_ref, sb_ref):
    scores0 = score_ref[...]
    lab_i = label_ref[...]

    px1 = p0_ref[...]
    py1 = p1_ref[...]
    px2 = p2_ref[...]
    py2 = p3_ref[...]
    w = px2 - px1 + 1.0
    h = py2 - py1 + 1.0
    cx = px1 + 0.5 * w
    cy = py1 + 0.5 * h
    dx = r0_ref[...] / 10.0
    dy = r1_ref[...] / 10.0
    dw = jnp.minimum(r2_ref[...] / 5.0, _CLIP)
    dh = jnp.minimum(r3_ref[...] / 5.0, _CLIP)
    pcx = dx * w + cx
    pcy = dy * h + cy
    pw = jnp.exp(dw) * w
    ph = jnp.exp(dh) * h
    bx1 = jnp.clip(pcx - 0.5 * pw, 0.0, _IMG_W - 1.0)
    by1 = jnp.clip(pcy - 0.5 * ph, 0.0, _IMG_H - 1.0)
    bx2 = jnp.clip(pcx + 0.5 * pw - 1.0, 0.0, _IMG_W - 1.0)
    by2 = jnp.clip(pcy + 0.5 * ph - 1.0, 0.0, _IMG_H - 1.0)
    areas = (bx2 - bx1 + 1.0) * (by2 - by1 + 1.0)

    sb_ref[0 * _ROWS:1 * _ROWS, :] = bx1
    sb_ref[1 * _ROWS:2 * _ROWS, :] = by1
    sb_ref[2 * _ROWS:3 * _ROWS, :] = bx2
    sb_ref[3 * _ROWS:4 * _ROWS, :] = by2
    sb_ref[4 * _ROWS:5 * _ROWS, :] = areas
    sb_ref[5 * _ROWS:6 * _ROWS, :] = lab_i.astype(jnp.float32)

    flat = (lax.broadcasted_iota(jnp.int32, (_ROWS, _LANES), 0) * _LANES
            + lax.broadcasted_iota(jnp.int32, (_ROWS, _LANES), 1))
    col = lax.broadcasted_iota(jnp.int32, (1, _LANES), 1)
    zrow = jnp.zeros((1, _LANES), jnp.float32)

    def step(i, carry):
        scores, os_, ox1, oy1, ox2, oy2, ol = carry
        gm = jnp.max(scores)
        bf = jnp.min(jnp.where(scores == gm, flat, jnp.int32(2147483647)))
        isb = flat == bf
        br = bf >> 7
        cm = (col == (bf & 127)).astype(jnp.float32)
        sx1 = jnp.sum(sb_ref[pl.ds(0 * _ROWS + br, 1), :] * cm)
        sy1 = jnp.sum(sb_ref[pl.ds(1 * _ROWS + br, 1), :] * cm)
        sx2 = jnp.sum(sb_ref[pl.ds(2 * _ROWS + br, 1), :] * cm)
        sy2 = jnp.sum(sb_ref[pl.ds(3 * _ROWS + br, 1), :] * cm)
        sarea = jnp.sum(sb_ref[pl.ds(4 * _ROWS + br, 1), :] * cm)
        slab = jnp.sum(sb_ref[pl.ds(5 * _ROWS + br, 1), :] * cm)
        xx1 = jnp.maximum(sx1, bx1)
        yy1 = jnp.maximum(sy1, by1)
        xx2 = jnp.minimum(sx2, bx2)
        yy2 = jnp.minimum(sy2, by2)
        inter = (jnp.maximum(xx2 - xx1 + 1.0, 0.0)
                 * jnp.maximum(yy2 - yy1 + 1.0, 0.0))
        iou = inter / (sarea + areas - inter)
        scores = jnp.where((iou > _NMS_THRESH) | isb, _NEG, scores)
        valid = gm > 0.0
        vf = jnp.where(valid, 1.0, 0.0)
        hit = col == i
        os_ = jnp.where(hit, gm * vf, os_)
        ox1 = jnp.where(hit, sx1 * vf, ox1)
        oy1 = jnp.where(hit, sy1 * vf, oy1)
        ox2 = jnp.where(hit, sx2 * vf, ox2)
        oy2 = jnp.where(hit, sy2 * vf, oy2)
        ol = jnp.where(hit, slab * vf, ol)
        return scores, os_, ox1, oy1, ox2, oy2, ol

    init = (scores0, zrow, zrow, zrow, zrow, zrow, zrow)
    _, os_, ox1, oy1, ox2, oy2, ol = lax.fori_loop(0, _DETS, step, init)
    obox_ref[0:1, :] = ox1
    obox_ref[1:2, :] = oy1
    obox_ref[2:3, :] = ox2
    obox_ref[3:4, :] = oy2
    oscore_ref[...] = os_
    olab_ref[...] = (ol + 0.5).astype(jnp.int32)


_nms_call = pl.pallas_call(
    _nms_body,
    out_shape=[
        jax.ShapeDtypeStruct((4, _LANES), jnp.float32),
        jax.ShapeDtypeStruct((1, _LANES), jnp.float32),
        jax.ShapeDtypeStruct((1, _LANES), jnp.int32),
    ],
    scratch_shapes=[pltpu.VMEM((6 * _ROWS, _LANES), jnp.float32)],
)


@jax.jit
def kernel(class_logits, box_regression, proposal_boxes):
    (score, label, i0, i1, i2, i3, table,
     p0, p1, p2, p3) = _score_call(class_logits.T, box_regression.T,
                                   proposal_boxes.T)
    r0, r1, r2, r3 = _gather_call(i0, i1, i2, i3, table.reshape(-1))
    obox, oscore, olab = _nms_call(score, label, r0, r1, r2, r3,
                                   p0, p1, p2, p3)
    return obox[:, :_DETS].T, oscore[0, :_DETS], olab[0, :_DETS]
